# baseline (speedup 1.0000x reference)
import functools

import jax
import jax.numpy as jnp
from jax import lax
from jax.experimental import pallas as pl
from jax.experimental.pallas import tpu as pltpu
from jax.experimental.pallas import tpu_sc as plsc

B = 16384
D = 300
NC = 2
NS = 16
NW = NC * NS
B_PER_W = B // NW
CHUNK = 128
N_CHUNKS = B_PER_W // CHUNK

BM = 512


def _sc_gather(src_id, tgt_id, emb_src, emb_tgt):
    mesh = plsc.VectorSubcoreMesh(
        core_axis_name="c", subcore_axis_name="s", num_cores=NC, num_subcores=NS
    )

    @functools.partial(
        pl.kernel,
        out_type=[
            jax.ShapeDtypeStruct((2 * B, D), jnp.float32),
            jax.ShapeDtypeStruct((B, D), jnp.float32),
        ],
        mesh=mesh,
        compiler_params=pltpu.CompilerParams(use_tc_tiling_on_sc=False),
        scratch_types=[
            pltpu.VMEM((CHUNK,), jnp.int32),
            pltpu.VMEM((CHUNK, D), jnp.float32),
            pltpu.SemaphoreType.DMA,
        ],
    )
    def k(src_id_hbm, tgt_id_hbm, src_tab, tgt_tab, big_out, src_rows_out,
          idx_v, rows_v, sem):
        wid = lax.axis_index("s") * NC + lax.axis_index("c")
        base = wid * B_PER_W
        for j in range(N_CHUNKS):
            off = base + j * CHUNK
            pltpu.sync_copy(src_id_hbm.at[pl.ds(off, CHUNK)], idx_v)
            pltpu.async_copy(src_tab.at[idx_v], rows_v, sem).wait()
            pltpu.sync_copy(rows_v, src_rows_out.at[pl.ds(off, CHUNK)])
        for j in range(N_CHUNKS):
            off = base + j * CHUNK
            pltpu.sync_copy(tgt_id_hbm.at[pl.ds(off, CHUNK)], idx_v)
            pltpu.async_copy(tgt_tab.at[idx_v], rows_v, sem).wait()
            pltpu.sync_copy(rows_v, big_out.at[pl.ds(B + off, CHUNK)])

    return k(src_id, tgt_id, emb_src, emb_tgt)


def _tc_map_kernel(x_ref, we_ref, b_ref, wd_ref, big_ref, out_ref, m_scr, c_scr):
    del big_ref

    @pl.when(pl.program_id(0) == 0)
    def _():
        m_scr[...] = lax.dot_general(
            we_ref[...], wd_ref[...],
            dimension_numbers=(((0,), (0,)), ((), ())),
            preferred_element_type=jnp.float32,
        )
        c_scr[...] = jnp.dot(b_ref[...], wd_ref[...],
                             preferred_element_type=jnp.float32)

    out_ref[...] = (
        jnp.dot(x_ref[...], m_scr[...], preferred_element_type=jnp.float32)
        + c_scr[...]
    )


def _tc_map(src_rows, W_enc, b_enc, W_dec, big):
    return pl.pallas_call(
        _tc_map_kernel,
        grid=(B // BM,),
        in_specs=[
            pl.BlockSpec((BM, D), lambda i: (i, 0)),
            pl.BlockSpec((D, D), lambda i: (0, 0)),
            pl.BlockSpec((1, D), lambda i: (0, 0)),
            pl.BlockSpec((D, D), lambda i: (0, 0)),
            pl.BlockSpec(memory_space=pl.ANY),
        ],
        out_specs=pl.BlockSpec((BM, D), lambda i: (i, 0)),
        out_shape=jax.ShapeDtypeStruct((2 * B, D), jnp.float32),
        scratch_shapes=[
            pltpu.VMEM((D, D), jnp.float32),
            pltpu.VMEM((1, D), jnp.float32),
        ],
        input_output_aliases={4: 0},
    )(src_rows, W_enc, b_enc, W_dec, big)


def kernel(src_id, tgt_id, emb_src, emb_tgt, W_enc, b_enc, W_dec):
    big, src_rows = _sc_gather(src_id.astype(jnp.int32), tgt_id.astype(jnp.int32),
                               emb_src, emb_tgt)
    return _tc_map(src_rows, W_enc, b_enc.reshape(1, D), W_dec, big)


# speedup vs baseline: 3.5285x; 3.5285x over previous
import functools

import jax
import jax.numpy as jnp
from jax import lax
from jax.experimental import pallas as pl
from jax.experimental.pallas import tpu as pltpu
from jax.experimental.pallas import tpu_sc as plsc

V = 100000
B = 16384
D = 300
DP = 384
NSLAB = 3
NC = 2
NS = 16
NW = NC * NS
B_PER_W = B // NW
CHUNK = 128
N_CHUNKS = B_PER_W // CHUNK

TBM = 512
TGRID = (V + TBM - 1) // TBM

BM = 512


def _slab_kernel(ts_ref, tt_ref, *out_refs):
    s = jnp.transpose(ts_ref[...], (1, 0))
    t = jnp.transpose(tt_ref[...], (1, 0))
    zpad = jnp.zeros((TBM, DP - D), jnp.float32)
    s = jnp.concatenate([s, zpad], axis=1)
    t = jnp.concatenate([t, zpad], axis=1)
    for k in range(NSLAB):
        out_refs[k][...] = s[:, k * 128:(k + 1) * 128]
        out_refs[NSLAB + k][...] = t[:, k * 128:(k + 1) * 128]


def _slabs(embT_src, embT_tgt):
    return pl.pallas_call(
        _slab_kernel,
        grid=(TGRID,),
        in_specs=[
            pl.BlockSpec((D, TBM), lambda i: (0, i)),
            pl.BlockSpec((D, TBM), lambda i: (0, i)),
        ],
        out_specs=[pl.BlockSpec((TBM, 128), lambda i: (i, 0))] * (2 * NSLAB),
        out_shape=[jax.ShapeDtypeStruct((V, 128), jnp.float32)] * (2 * NSLAB),
    )(embT_src, embT_tgt)


def _sc_gather(ids3, s1, s2, s3, t1, t2, t3):
    mesh = plsc.VectorSubcoreMesh(
        core_axis_name="c", subcore_axis_name="s", num_cores=NC, num_subcores=NS
    )

    @functools.partial(
        pl.kernel,
        out_type=[jax.ShapeDtypeStruct((2 * B, 128), jnp.float32)] * NSLAB,
        mesh=mesh,
        scratch_types=[
            pltpu.VMEM((CHUNK,), jnp.int32),
            pltpu.VMEM((CHUNK, 128), jnp.float32),
            pltpu.VMEM((CHUNK, 128), jnp.float32),
            pltpu.VMEM((CHUNK, 128), jnp.float32),
            pltpu.SemaphoreType.DMA,
        ],
    )
    def k(ids_hbm, s1h, s2h, s3h, t1h, t2h, t3h, x1, x2, x3,
          idx_v, r1, r2, r3, sem):
        wid = lax.axis_index("s") * NC + lax.axis_index("c")
        base = wid * B_PER_W
        rbufs = (r1, r2, r3)
        outs = (x1, x2, x3)
        for half, tabs in enumerate(((s1h, s2h, s3h), (t1h, t2h, t3h))):
            for j in range(N_CHUNKS):
                off = base + j * CHUNK
                pltpu.sync_copy(ids_hbm.at[half, wid, j], idx_v)
                cps = [pltpu.async_copy(tabs[k_].at[idx_v], rbufs[k_], sem)
                       for k_ in range(NSLAB)]
                for cp in cps:
                    cp.wait()
                dst = half * B + off
                for k_ in range(NSLAB):
                    pltpu.sync_copy(rbufs[k_], outs[k_].at[pl.ds(dst, CHUNK)])

    return k(ids3, s1, s2, s3, t1, t2, t3)


def _map_kernel(x1_ref, x2_ref, x3_ref, we_ref, b_ref, wd_ref, out_ref,
                m_scr, c_scr):
    i = pl.program_id(0)

    @pl.when(i == 0)
    def _():
        m = lax.dot_general(
            we_ref[...], wd_ref[...],
            dimension_numbers=(((0,), (0,)), ((), ())),
            preferred_element_type=jnp.float32,
        )
        m_scr[...] = jnp.concatenate(
            [m, jnp.zeros((DP - D, D), jnp.float32)], axis=0)
        c_scr[...] = jnp.dot(b_ref[...], wd_ref[...],
                             preferred_element_type=jnp.float32)

    x = jnp.concatenate([x1_ref[...], x2_ref[...], x3_ref[...]], axis=1)

    @pl.when(i < B // BM)
    def _():
        out_ref[...] = (
            jnp.dot(x, m_scr[...], preferred_element_type=jnp.float32)
            + c_scr[...]
        )

    @pl.when(i >= B // BM)
    def _():
        out_ref[...] = x[:, :D]


def _tc_map(x1, x2, x3, W_enc, b_enc, W_dec):
    return pl.pallas_call(
        _map_kernel,
        grid=(2 * B // BM,),
        in_specs=[
            pl.BlockSpec((BM, 128), lambda i: (i, 0)),
            pl.BlockSpec((BM, 128), lambda i: (i, 0)),
            pl.BlockSpec((BM, 128), lambda i: (i, 0)),
            pl.BlockSpec((D, D), lambda i: (0, 0)),
            pl.BlockSpec((1, D), lambda i: (0, 0)),
            pl.BlockSpec((D, D), lambda i: (0, 0)),
        ],
        out_specs=pl.BlockSpec((BM, D), lambda i: (i, 0)),
        out_shape=jax.ShapeDtypeStruct((2 * B, D), jnp.float32),
        scratch_shapes=[
            pltpu.VMEM((DP, D), jnp.float32),
            pltpu.VMEM((1, D), jnp.float32),
        ],
    )(x1, x2, x3, W_enc, b_enc, W_dec)


def kernel(src_id, tgt_id, emb_src, emb_tgt, W_enc, b_enc, W_dec):
    slabs = _slabs(emb_src.T, emb_tgt.T)
    ids3 = jnp.stack([src_id.astype(jnp.int32), tgt_id.astype(jnp.int32)]
                     ).reshape(2, NW, N_CHUNKS, CHUNK)
    x1, x2, x3 = _sc_gather(ids3, *slabs)
    return _tc_map(x1, x2, x3, W_enc, b_enc.reshape(1, D), W_dec)


# speedup vs baseline: 3.6738x; 1.0412x over previous
import functools

import jax
import jax.numpy as jnp
from jax import lax
from jax.experimental import pallas as pl
from jax.experimental.pallas import tpu as pltpu
from jax.experimental.pallas import tpu_sc as plsc

V = 100000
B = 16384
D = 300
DPACK = 256
NSLAB = 2
NC = 2
NS = 16
NW = NC * NS
B_PER_W = B // NW
CHUNK = 128
N_CHUNKS = B_PER_W // CHUNK

TBM = 512
TGRID = (V + TBM - 1) // TBM

BM = 512


HALF = 150


def _pack(x):
    lo = lax.bitcast_convert_type(x[:, :HALF], jnp.uint32)
    hi = lax.bitcast_convert_type(x[:, HALF:], jnp.uint32)
    rnd = jnp.uint32(0x8000)
    w = ((lo + rnd) >> 16) | ((hi + rnd) & jnp.uint32(0xFFFF0000))
    w = jnp.concatenate(
        [w, jnp.zeros((x.shape[0], DPACK - HALF), jnp.uint32)], axis=1)
    return lax.bitcast_convert_type(w, jnp.float32)


def _slab_kernel(ts_ref, tt_ref, s1_ref, s2_ref, t1_ref, t2_ref):
    s = _pack(jnp.transpose(ts_ref[...], (1, 0)))
    t = _pack(jnp.transpose(tt_ref[...], (1, 0)))
    s1_ref[...] = s[:, :128]
    s2_ref[...] = s[:, 128:]
    t1_ref[...] = t[:, :128]
    t2_ref[...] = t[:, 128:]


def _slabs(embT_src, embT_tgt):
    return pl.pallas_call(
        _slab_kernel,
        grid=(TGRID,),
        in_specs=[
            pl.BlockSpec((D, TBM), lambda i: (0, i)),
            pl.BlockSpec((D, TBM), lambda i: (0, i)),
        ],
        out_specs=[pl.BlockSpec((TBM, 128), lambda i: (i, 0))] * (2 * NSLAB),
        out_shape=[jax.ShapeDtypeStruct((V, 128), jnp.float32)] * (2 * NSLAB),
    )(embT_src, embT_tgt)


def _sc_gather(ids3, s1, s2, t1, t2):
    mesh = plsc.VectorSubcoreMesh(
        core_axis_name="c", subcore_axis_name="s", num_cores=NC, num_subcores=NS
    )

    @functools.partial(
        pl.kernel,
        out_type=[jax.ShapeDtypeStruct((2 * B, 128), jnp.float32)] * NSLAB,
        mesh=mesh,
        scratch_types=[
            pltpu.VMEM((CHUNK,), jnp.int32),
            pltpu.VMEM((CHUNK, 128), jnp.float32),
            pltpu.VMEM((CHUNK, 128), jnp.float32),
            pltpu.SemaphoreType.DMA,
        ],
    )
    def k(ids_hbm, s1h, s2h, t1h, t2h, x1, x2, idx_v, r1, r2, sem):
        wid = lax.axis_index("s") * NC + lax.axis_index("c")
        base = wid * B_PER_W
        rbufs = (r1, r2)
        outs = (x1, x2)
        for half, tabs in enumerate(((s1h, s2h), (t1h, t2h))):
            for j in range(N_CHUNKS):
                off = base + j * CHUNK
                pltpu.sync_copy(ids_hbm.at[half, wid, j], idx_v)
                cps = [pltpu.async_copy(tabs[k_].at[idx_v], rbufs[k_], sem)
                       for k_ in range(NSLAB)]
                for cp in cps:
                    cp.wait()
                dst = half * B + off
                for k_ in range(NSLAB):
                    pltpu.sync_copy(rbufs[k_], outs[k_].at[pl.ds(dst, CHUNK)])

    return k(ids3, s1, s2, t1, t2)


def _unpack(x1, x2):
    p = lax.bitcast_convert_type(
        jnp.concatenate([x1, x2], axis=1)[:, :HALF], jnp.uint32)
    lo = lax.bitcast_convert_type(p << 16, jnp.float32)
    hi = lax.bitcast_convert_type(p & jnp.uint32(0xFFFF0000), jnp.float32)
    return jnp.concatenate([lo, hi], axis=1)


def _map_kernel(x1_ref, x2_ref, we_ref, b_ref, wd_ref, out_ref, m_scr, c_scr):
    i = pl.program_id(0)

    @pl.when(i == 0)
    def _():
        m_scr[...] = lax.dot_general(
            we_ref[...], wd_ref[...],
            dimension_numbers=(((0,), (0,)), ((), ())),
            preferred_element_type=jnp.float32,
        )
        c_scr[...] = jnp.dot(b_ref[...], wd_ref[...],
                             preferred_element_type=jnp.float32)

    x = _unpack(x1_ref[...], x2_ref[...])

    @pl.when(i < B // BM)
    def _():
        out_ref[...] = (
            jnp.dot(x, m_scr[...], preferred_element_type=jnp.float32)
            + c_scr[...]
        )

    @pl.when(i >= B // BM)
    def _():
        out_ref[...] = x


def _tc_map(x1, x2, W_enc, b_enc, W_dec):
    return pl.pallas_call(
        _map_kernel,
        grid=(2 * B // BM,),
        in_specs=[
            pl.BlockSpec((BM, 128), lambda i: (i, 0)),
            pl.BlockSpec((BM, 128), lambda i: (i, 0)),
            pl.BlockSpec((D, D), lambda i: (0, 0)),
            pl.BlockSpec((1, D), lambda i: (0, 0)),
            pl.BlockSpec((D, D), lambda i: (0, 0)),
        ],
        out_specs=pl.BlockSpec((BM, D), lambda i: (i, 0)),
        out_shape=jax.ShapeDtypeStruct((2 * B, D), jnp.float32),
        scratch_shapes=[
            pltpu.VMEM((D, D), jnp.float32),
            pltpu.VMEM((1, D), jnp.float32),
        ],
    )(x1, x2, W_enc, b_enc, W_dec)


def kernel(src_id, tgt_id, emb_src, emb_tgt, W_enc, b_enc, W_dec):
    s1, s2, t1, t2 = _slabs(emb_src.T, emb_tgt.T)
    ids3 = jnp.stack([src_id.astype(jnp.int32), tgt_id.astype(jnp.int32)]
                     ).reshape(2, NW, N_CHUNKS, CHUNK)
    x1, x2 = _sc_gather(ids3, s1, s2, t1, t2)
    return _tc_map(x1, x2, W_enc, b_enc.reshape(1, D), W_dec)


# speedup vs baseline: 4.2435x; 1.1551x over previous
import functools

import jax
import jax.numpy as jnp
from jax import lax
from jax.experimental import pallas as pl
from jax.experimental.pallas import tpu as pltpu
from jax.experimental.pallas import tpu_sc as plsc

V = 100000
B = 16384
D = 300
HALF = 150
DPACK = 256
NSLAB = 2
NC = 2
NS = 16
NW = NC * NS
B_PER_W = B // NW
CHUNK = 128
N_CHUNKS = B_PER_W // CHUNK

TBM = 1024
TGRID = (V + TBM - 1) // TBM

BM = 512


def _pack(x):
    lo = lax.bitcast_convert_type(x[:, :HALF], jnp.uint32)
    hi = lax.bitcast_convert_type(x[:, HALF:], jnp.uint32)
    rnd = jnp.uint32(0x8000)
    w = ((lo + rnd) >> 16) | ((hi + rnd) & jnp.uint32(0xFFFF0000))
    w = jnp.concatenate(
        [w, jnp.zeros((x.shape[0], DPACK - HALF), jnp.uint32)], axis=1)
    return lax.bitcast_convert_type(w, jnp.float32)


def _slab_kernel(ts_ref, tt_ref, we_ref, b_ref, wd_ref,
                 s1_ref, s2_ref, t1_ref, t2_ref, m_scr, i_scr, c_scr):
    @pl.when(pl.program_id(0) == 0)
    def _():
        m = lax.dot_general(
            we_ref[...], wd_ref[...],
            dimension_numbers=(((0,), (0,)), ((), ())),
            preferred_element_type=jnp.float32,
        )
        m_scr[...] = m.astype(jnp.bfloat16)
        i_scr[...] = (
            lax.broadcasted_iota(jnp.int32, (D, D), 0)
            == lax.broadcasted_iota(jnp.int32, (D, D), 1)
        ).astype(jnp.bfloat16)
        c_scr[...] = jnp.dot(b_ref[...], wd_ref[...],
                             preferred_element_type=jnp.float32)

    zs = lax.dot_general(
        ts_ref[...].astype(jnp.bfloat16), m_scr[...],
        dimension_numbers=(((0,), (0,)), ((), ())),
        preferred_element_type=jnp.float32,
    ) + c_scr[...]
    zt = lax.dot_general(
        tt_ref[...].astype(jnp.bfloat16), i_scr[...],
        dimension_numbers=(((0,), (0,)), ((), ())),
        preferred_element_type=jnp.float32,
    )
    s = _pack(zs)
    t = _pack(zt)
    s1_ref[...] = s[:, :128]
    s2_ref[...] = s[:, 128:]
    t1_ref[...] = t[:, :128]
    t2_ref[...] = t[:, 128:]


def _slabs(embT_src, embT_tgt, W_enc, b_enc, W_dec):
    return pl.pallas_call(
        _slab_kernel,
        grid=(TGRID,),
        in_specs=[
            pl.BlockSpec((D, TBM), lambda i: (0, i)),
            pl.BlockSpec((D, TBM), lambda i: (0, i)),
            pl.BlockSpec((D, D), lambda i: (0, 0)),
            pl.BlockSpec((1, D), lambda i: (0, 0)),
            pl.BlockSpec((D, D), lambda i: (0, 0)),
        ],
        out_specs=[pl.BlockSpec((TBM, 128), lambda i: (i, 0))] * (2 * NSLAB),
        out_shape=[jax.ShapeDtypeStruct((V, 128), jnp.float32)] * (2 * NSLAB),
        scratch_shapes=[
            pltpu.VMEM((D, D), jnp.bfloat16),
            pltpu.VMEM((D, D), jnp.bfloat16),
            pltpu.VMEM((1, D), jnp.float32),
        ],
    )(embT_src, embT_tgt, W_enc, b_enc, W_dec)


def _sc_gather(ids3, s1, s2, t1, t2):
    mesh = plsc.VectorSubcoreMesh(
        core_axis_name="c", subcore_axis_name="s", num_cores=NC, num_subcores=NS
    )

    @functools.partial(
        pl.kernel,
        out_type=[jax.ShapeDtypeStruct((2 * B, 128), jnp.float32)] * NSLAB,
        mesh=mesh,
        scratch_types=[
            pltpu.VMEM((CHUNK,), jnp.int32),
            pltpu.VMEM((CHUNK, 128), jnp.float32),
            pltpu.VMEM((CHUNK, 128), jnp.float32),
            pltpu.SemaphoreType.DMA,
        ],
    )
    def k(ids_hbm, s1h, s2h, t1h, t2h, x1, x2, idx_v, r1, r2, sem):
        wid = lax.axis_index("s") * NC + lax.axis_index("c")
        base = wid * B_PER_W
        rbufs = (r1, r2)
        outs = (x1, x2)
        for half, tabs in enumerate(((s1h, s2h), (t1h, t2h))):
            for j in range(N_CHUNKS):
                off = base + j * CHUNK
                pltpu.sync_copy(ids_hbm.at[half, wid, j], idx_v)
                cps = [pltpu.async_copy(tabs[k_].at[idx_v], rbufs[k_], sem)
                       for k_ in range(NSLAB)]
                for cp in cps:
                    cp.wait()
                dst = half * B + off
                for k_ in range(NSLAB):
                    pltpu.sync_copy(rbufs[k_], outs[k_].at[pl.ds(dst, CHUNK)])

    return k(ids3, s1, s2, t1, t2)


def _unpack(x1, x2):
    p = lax.bitcast_convert_type(
        jnp.concatenate([x1, x2], axis=1)[:, :HALF], jnp.uint32)
    lo = lax.bitcast_convert_type(p << 16, jnp.float32)
    hi = lax.bitcast_convert_type(p & jnp.uint32(0xFFFF0000), jnp.float32)
    return jnp.concatenate([lo, hi], axis=1)


def _map_kernel(x1_ref, x2_ref, out_ref):
    out_ref[...] = _unpack(x1_ref[...], x2_ref[...])


def _tc_map(x1, x2):
    return pl.pallas_call(
        _map_kernel,
        grid=(2 * B // BM,),
        in_specs=[
            pl.BlockSpec((BM, 128), lambda i: (i, 0)),
            pl.BlockSpec((BM, 128), lambda i: (i, 0)),
        ],
        out_specs=pl.BlockSpec((BM, D), lambda i: (i, 0)),
        out_shape=jax.ShapeDtypeStruct((2 * B, D), jnp.float32),
    )(x1, x2)


def kernel(src_id, tgt_id, emb_src, emb_tgt, W_enc, b_enc, W_dec):
    s1, s2, t1, t2 = _slabs(emb_src.T, emb_tgt.T, W_enc, b_enc.reshape(1, D),
                            W_dec)
    ids3 = jnp.stack([src_id.astype(jnp.int32), tgt_id.astype(jnp.int32)]
                     ).reshape(2, NW, N_CHUNKS, CHUNK)
    x1, x2 = _sc_gather(ids3, s1, s2, t1, t2)
    return _tc_map(x1, x2)
